# baseline (speedup 1.0000x reference)
import jax
import jax.numpy as jnp
from jax import lax
from jax.experimental import pallas as pl
from jax.experimental.pallas import tpu as pltpu
from jax.experimental.pallas import tpu_sc as plsc

_NW = 32
_CHUNK_ROWS = 400
_UNROLL = 16
_N_BLOCK = 2000


def _sc_colsum_body(x1_hbm, out_hbm, buf0, buf1, outv, sem0, sem1):
    rows, h = x1_hbm.shape
    rows_per_w = rows // _NW
    n_chunks = rows_per_w // _CHUNK_ROWS
    chunk_elts = _CHUNK_ROWS * h

    wid = lax.axis_index("s") * 2 + lax.axis_index("c")
    base = wid * rows_per_w

    bufs = (buf0, buf1)
    sems = (sem0, sem1)
    copies = [
        pltpu.async_copy(x1_hbm.at[pl.ds(base, _CHUNK_ROWS)], buf0, sem0),
        None,
    ]

    zero = jnp.zeros((16,), jnp.float32)
    accs = tuple(zero for _ in range(_UNROLL))
    n_inner = _CHUNK_ROWS // (_UNROLL // 2)

    for t in range(n_chunks):
        b = t % 2
        if t + 1 < n_chunks:
            nb = (t + 1) % 2
            copies[nb] = pltpu.async_copy(
                x1_hbm.at[pl.ds(base + (t + 1) * _CHUNK_ROWS, _CHUNK_ROWS)],
                bufs[nb], sems[nb])
        copies[b].wait()
        buf = bufs[b]

        def body(k, acc, buf=buf):
            row0 = k * (_UNROLL // 2)
            new = []
            for u in range(_UNROLL // 2):
                new.append(acc[2 * u] + buf[row0 + u, pl.ds(0, 16)])
                new.append(acc[2 * u + 1] + buf[row0 + u, pl.ds(16, 16)])
            return tuple(new)

        accs = lax.fori_loop(0, n_inner, body, accs)

    lo = accs[0]
    hi = accs[1]
    for u in range(2, _UNROLL, 2):
        lo = lo + accs[u]
        hi = hi + accs[u + 1]
    outv[pl.ds(0, 16)] = lo
    outv[pl.ds(16, 16)] = hi
    pltpu.sync_copy(outv, out_hbm.at[pl.ds(wid * h, h)])


def _sc_colsum(x_1):
    e, h = x_1.shape
    mesh = plsc.VectorSubcoreMesh(core_axis_name="c", subcore_axis_name="s",
                                  num_cores=2, num_subcores=16)
    sums_flat = pl.kernel(
        _sc_colsum_body,
        out_type=jax.ShapeDtypeStruct((_NW * h,), jnp.float32),
        mesh=mesh,
        compiler_params=pltpu.CompilerParams(use_tc_tiling_on_sc=False),
        scratch_types=[
            pltpu.VMEM((_CHUNK_ROWS, h), jnp.float32),
            pltpu.VMEM((_CHUNK_ROWS, h), jnp.float32),
            pltpu.VMEM((h,), jnp.float32),
            pltpu.SemaphoreType.DMA,
            pltpu.SemaphoreType.DMA,
        ],
    )(x_1)
    return sums_flat.reshape(_NW, h)


def _dense_body(s_ref, x0_ref, w0cat_ref, w1on_ref, w1tg_ref, p1w_ref,
                p1b_ref, a_ref, p2w_ref, p2b_ref, inv_e_ref,
                on_ref, pred_ref, tg_ref):
    def dot_t(a, b):
        return lax.dot_general(a, b, (((1,), (1,)), ((), ())),
                               preferred_element_type=jnp.float32)

    m32 = jnp.sum(s_ref[...], axis=0, keepdims=True) * inv_e_ref[0, 0]
    m_on = dot_t(m32, w1on_ref[...])
    m_tg = dot_t(m32, w1tg_ref[...])

    x0 = x0_ref[...]
    x0_cat = dot_t(x0, w0cat_ref[...])
    n = x0.shape[0]
    h_on = jnp.concatenate(
        [x0_cat[:, 0:32], jnp.broadcast_to(m_on, (n, 32))], axis=1)
    h_tg = jnp.concatenate(
        [x0_cat[:, 32:64], jnp.broadcast_to(m_tg, (n, 32))], axis=1)
    on_ref[...] = h_on
    tg_ref[...] = h_tg

    z = dot_t(h_on, p1w_ref[...]) + p1b_ref[...]
    a = a_ref[0, 0]
    h = jnp.where(z >= 0, z, a * z)
    pred_ref[...] = dot_t(h, p2w_ref[...]) + p2b_ref[...]


def kernel(x_0, x_1, adjacency_0, down_laplacian, up_laplacian,
           W0_on, W1_on, W0_tg, W1_tg, p1_w, p1_b, prelu_a, p2_w, p2_b):
    n, in0 = x_0.shape
    e, h = x_1.shape

    sums = _sc_colsum(x_1)

    w0cat = jnp.concatenate([W0_on, W0_tg], axis=0)
    p1b = p1_b.reshape(1, h)
    p2b = p2_b.reshape(1, 2 * h)
    a = jnp.reshape(prelu_a, (1, 1))
    inv_e = jnp.full((1, 1), 1.0 / e, dtype=jnp.float32)

    const = lambda shape: pl.BlockSpec(shape, lambda i: tuple(0 for _ in shape))
    n_blocks = n // _N_BLOCK
    h_on, h_pred, h_tg = pl.pallas_call(
        _dense_body,
        grid=(n_blocks,),
        in_specs=[
            const((_NW, h)),
            pl.BlockSpec((_N_BLOCK, in0), lambda i: (i, 0)),
            const((2 * h, in0)),
            const((h, h)),
            const((h, h)),
            const((h, 2 * h)),
            const((1, h)),
            const((1, 1)),
            const((2 * h, h)),
            const((1, 2 * h)),
            const((1, 1)),
        ],
        out_specs=[
            pl.BlockSpec((_N_BLOCK, 2 * h), lambda i: (i, 0)),
            pl.BlockSpec((_N_BLOCK, 2 * h), lambda i: (i, 0)),
            pl.BlockSpec((_N_BLOCK, 2 * h), lambda i: (i, 0)),
        ],
        out_shape=[
            jax.ShapeDtypeStruct((n, 2 * h), jnp.float32),
            jax.ShapeDtypeStruct((n, 2 * h), jnp.float32),
            jax.ShapeDtypeStruct((n, 2 * h), jnp.float32),
        ],
    )(sums, x_0, w0cat, W1_on, W1_tg, p1_w, p1b, a, p2_w, p2b, inv_e)

    return (h_on, h_pred, h_tg)


# speedup vs baseline: 6.8007x; 6.8007x over previous
import jax
import jax.numpy as jnp
from jax import lax
from jax.experimental import pallas as pl
from jax.experimental.pallas import tpu as pltpu

_E_BLOCK = 16000
_N_BLOCK = 4096


def _colsum_body(x1t_ref, out_ref):
    i = pl.program_id(0)

    @pl.when(i == 0)
    def _init():
        out_ref[...] = jnp.zeros_like(out_ref)

    x = x1t_ref[...]
    xr = x.reshape(x.shape[0], _E_BLOCK // 128, 128)
    out_ref[...] += jnp.sum(xr, axis=1)


def _dense_body(s_ref, x0_ref, w0cat_ref, w1on_ref, w1tg_ref, p1w_ref,
                p1b_ref, a_ref, p2w_ref, p2b_ref, inv_e_ref,
                on_ref, pred_ref, tg_ref):
    def mm(a, b):
        return lax.dot_general(a, b, (((1,), (0,)), ((), ())),
                               preferred_element_type=jnp.float32)

    def mm_t(a, b):
        return lax.dot_general(a, b, (((1,), (1,)), ((), ())),
                               preferred_element_type=jnp.float32)

    m = jnp.sum(s_ref[...], axis=1, keepdims=True) * inv_e_ref[0, 0]
    m_on = mm(w1on_ref[...], m)
    m_tg = mm(w1tg_ref[...], m)

    x0 = x0_ref[...]
    x0_cat_t = mm_t(w0cat_ref[...], x0)
    bn = x0.shape[0]
    h_on_t = jnp.concatenate(
        [x0_cat_t[0:32, :], jnp.broadcast_to(m_on, (32, bn))], axis=0)
    h_tg_t = jnp.concatenate(
        [x0_cat_t[32:64, :], jnp.broadcast_to(m_tg, (32, bn))], axis=0)
    on_ref[...] = h_on_t
    tg_ref[...] = h_tg_t

    z = mm(p1w_ref[...], h_on_t) + p1b_ref[...]
    a = a_ref[0, 0]
    h = jnp.where(z >= 0, z, a * z)
    pred_ref[...] = mm(p2w_ref[...], h) + p2b_ref[...]


def kernel(x_0, x_1, adjacency_0, down_laplacian, up_laplacian,
           W0_on, W1_on, W0_tg, W1_tg, p1_w, p1_b, prelu_a, p2_w, p2_b):
    n, in0 = x_0.shape
    e, h = x_1.shape

    x1t = x_1.T

    sums = pl.pallas_call(
        _colsum_body,
        grid=(e // _E_BLOCK,),
        in_specs=[pl.BlockSpec((h, _E_BLOCK), lambda i: (0, i))],
        out_specs=pl.BlockSpec((h, 128), lambda i: (0, 0)),
        out_shape=jax.ShapeDtypeStruct((h, 128), jnp.float32),
    )(x1t)

    w0cat = jnp.concatenate([W0_on, W0_tg], axis=0)
    p1b = p1_b.reshape(h, 1)
    p2b = p2_b.reshape(2 * h, 1)
    a = jnp.reshape(prelu_a, (1, 1))
    inv_e = jnp.full((1, 1), 1.0 / e, dtype=jnp.float32)

    const = lambda shape: pl.BlockSpec(shape, lambda i: tuple(0 for _ in shape))
    n_blocks = (n + _N_BLOCK - 1) // _N_BLOCK
    on_t, pred_t, tg_t = pl.pallas_call(
        _dense_body,
        grid=(n_blocks,),
        in_specs=[
            const((h, 128)),
            pl.BlockSpec((_N_BLOCK, in0), lambda i: (i, 0)),
            const((2 * h, in0)),
            const((h, h)),
            const((h, h)),
            const((h, 2 * h)),
            const((h, 1)),
            const((1, 1)),
            const((2 * h, h)),
            const((2 * h, 1)),
            const((1, 1)),
        ],
        out_specs=[
            pl.BlockSpec((2 * h, _N_BLOCK), lambda i: (0, i)),
            pl.BlockSpec((2 * h, _N_BLOCK), lambda i: (0, i)),
            pl.BlockSpec((2 * h, _N_BLOCK), lambda i: (0, i)),
        ],
        out_shape=[
            jax.ShapeDtypeStruct((2 * h, n), jnp.float32),
            jax.ShapeDtypeStruct((2 * h, n), jnp.float32),
            jax.ShapeDtypeStruct((2 * h, n), jnp.float32),
        ],
    )(sums, x_0, w0cat, W1_on, W1_tg, p1_w, p1b, a, p2_w, p2b, inv_e)

    return (on_t.T, pred_t.T, tg_t.T)


# speedup vs baseline: 8.4902x; 1.2484x over previous
import jax
import jax.numpy as jnp
from jax import lax
from jax.experimental import pallas as pl
from jax.experimental.pallas import tpu as pltpu

_E_BLOCK = 32000
_N_BLOCK = 4096


def _colsum_body(x1t_ref, out_ref):
    i = pl.program_id(0)

    @pl.when(i == 0)
    def _init():
        out_ref[...] = jnp.zeros_like(out_ref)

    x = x1t_ref[...]
    xr = x.reshape(x.shape[0], _E_BLOCK // 128, 128)
    out_ref[...] += jnp.sum(xr, axis=1)


def _dense_body(s_ref, x0_ref, w0cat_ref, w1on_ref, w1tg_ref, p1w_ref,
                p1b_ref, a_ref, p2w_ref, p2b_ref, inv_e_ref,
                on_ref, pred_ref, tg_ref):
    def mm(a, b):
        return lax.dot_general(a, b, (((1,), (0,)), ((), ())),
                               preferred_element_type=jnp.float32)

    def mm_t(a, b):
        return lax.dot_general(a, b, (((1,), (1,)), ((), ())),
                               preferred_element_type=jnp.float32)

    m = jnp.sum(s_ref[...], axis=1, keepdims=True) * inv_e_ref[0, 0]
    m_on = mm(w1on_ref[...], m)
    m_tg = mm(w1tg_ref[...], m)

    x0 = x0_ref[...]
    x0_cat_t = mm_t(w0cat_ref[...], x0)
    bn = x0.shape[0]
    h_on_t = jnp.concatenate(
        [x0_cat_t[0:32, :], jnp.broadcast_to(m_on, (32, bn))], axis=0)
    h_tg_t = jnp.concatenate(
        [x0_cat_t[32:64, :], jnp.broadcast_to(m_tg, (32, bn))], axis=0)
    on_ref[...] = h_on_t
    tg_ref[...] = h_tg_t

    z = mm(p1w_ref[...], h_on_t) + p1b_ref[...]
    a = a_ref[0, 0]
    h = jnp.where(z >= 0, z, a * z)
    pred_ref[...] = mm(p2w_ref[...], h) + p2b_ref[...]


def kernel(x_0, x_1, adjacency_0, down_laplacian, up_laplacian,
           W0_on, W1_on, W0_tg, W1_tg, p1_w, p1_b, prelu_a, p2_w, p2_b):
    n, in0 = x_0.shape
    e, h = x_1.shape

    x1t = x_1.T

    sums = pl.pallas_call(
        _colsum_body,
        grid=(e // _E_BLOCK,),
        in_specs=[pl.BlockSpec((h, _E_BLOCK), lambda i: (0, i))],
        out_specs=pl.BlockSpec((h, 128), lambda i: (0, 0)),
        out_shape=jax.ShapeDtypeStruct((h, 128), jnp.float32),
    )(x1t)

    w0cat = jnp.concatenate([W0_on, W0_tg], axis=0)
    p1b = p1_b.reshape(h, 1)
    p2b = p2_b.reshape(2 * h, 1)
    a = jnp.reshape(prelu_a, (1, 1))
    inv_e = jnp.full((1, 1), 1.0 / e, dtype=jnp.float32)

    const = lambda shape: pl.BlockSpec(shape, lambda i: tuple(0 for _ in shape))
    n_blocks = (n + _N_BLOCK - 1) // _N_BLOCK
    on_t, pred_t, tg_t = pl.pallas_call(
        _dense_body,
        grid=(n_blocks,),
        in_specs=[
            const((h, 128)),
            pl.BlockSpec((_N_BLOCK, in0), lambda i: (i, 0)),
            const((2 * h, in0)),
            const((h, h)),
            const((h, h)),
            const((h, 2 * h)),
            const((h, 1)),
            const((1, 1)),
            const((2 * h, h)),
            const((2 * h, 1)),
            const((1, 1)),
        ],
        out_specs=[
            pl.BlockSpec((2 * h, _N_BLOCK), lambda i: (0, i)),
            pl.BlockSpec((2 * h, _N_BLOCK), lambda i: (0, i)),
            pl.BlockSpec((2 * h, _N_BLOCK), lambda i: (0, i)),
        ],
        out_shape=[
            jax.ShapeDtypeStruct((2 * h, n), jnp.float32),
            jax.ShapeDtypeStruct((2 * h, n), jnp.float32),
            jax.ShapeDtypeStruct((2 * h, n), jnp.float32),
        ],
    )(sums, x_0, w0cat, W1_on, W1_tg, p1_w, p1b, a, p2_w, p2b, inv_e)

    return (on_t.T, pred_t.T, tg_t.T)


# speedup vs baseline: 9.6480x; 1.1364x over previous
import jax
import jax.numpy as jnp
from jax import lax
from jax.experimental import pallas as pl
from jax.experimental.pallas import tpu as pltpu

_E_BLOCK = 64000
_N_BLOCK = 4096


def _colsum_body(x1t_ref, out_ref):
    i = pl.program_id(0)

    @pl.when(i == 0)
    def _init():
        out_ref[...] = jnp.zeros_like(out_ref)

    x = x1t_ref[...]
    xr = x.reshape(x.shape[0], _E_BLOCK // 128, 128)
    out_ref[...] += jnp.sum(xr, axis=1)


def _dense_body(s_ref, x0_ref, w0cat_ref, w1on_ref, w1tg_ref, p1w_ref,
                p1b_ref, a_ref, p2w_ref, p2b_ref, inv_e_ref,
                on_ref, pred_ref, tg_ref):
    def mm(a, b):
        return lax.dot_general(a, b, (((1,), (0,)), ((), ())),
                               preferred_element_type=jnp.float32)

    def mm_t(a, b):
        return lax.dot_general(a, b, (((1,), (1,)), ((), ())),
                               preferred_element_type=jnp.float32)

    m = jnp.sum(s_ref[...], axis=1, keepdims=True) * inv_e_ref[0, 0]
    m_on = mm(w1on_ref[...], m)
    m_tg = mm(w1tg_ref[...], m)

    x0 = x0_ref[...]
    x0_cat_t = mm_t(w0cat_ref[...], x0)
    bn = x0.shape[0]
    h_on_t = jnp.concatenate(
        [x0_cat_t[0:32, :], jnp.broadcast_to(m_on, (32, bn))], axis=0)
    h_tg_t = jnp.concatenate(
        [x0_cat_t[32:64, :], jnp.broadcast_to(m_tg, (32, bn))], axis=0)
    on_ref[...] = h_on_t
    tg_ref[...] = h_tg_t

    z = mm(p1w_ref[...], h_on_t) + p1b_ref[...]
    a = a_ref[0, 0]
    h = jnp.where(z >= 0, z, a * z)
    pred_ref[...] = mm(p2w_ref[...], h) + p2b_ref[...]


def kernel(x_0, x_1, adjacency_0, down_laplacian, up_laplacian,
           W0_on, W1_on, W0_tg, W1_tg, p1_w, p1_b, prelu_a, p2_w, p2_b):
    n, in0 = x_0.shape
    e, h = x_1.shape

    x1t = x_1.T

    sums = pl.pallas_call(
        _colsum_body,
        grid=(e // _E_BLOCK,),
        in_specs=[pl.BlockSpec((h, _E_BLOCK), lambda i: (0, i))],
        out_specs=pl.BlockSpec((h, 128), lambda i: (0, 0)),
        out_shape=jax.ShapeDtypeStruct((h, 128), jnp.float32),
    )(x1t)

    w0cat = jnp.concatenate([W0_on, W0_tg], axis=0)
    p1b = p1_b.reshape(h, 1)
    p2b = p2_b.reshape(2 * h, 1)
    a = jnp.reshape(prelu_a, (1, 1))
    inv_e = jnp.full((1, 1), 1.0 / e, dtype=jnp.float32)

    const = lambda shape: pl.BlockSpec(shape, lambda i: tuple(0 for _ in shape))
    n_blocks = (n + _N_BLOCK - 1) // _N_BLOCK
    on_t, pred_t, tg_t = pl.pallas_call(
        _dense_body,
        grid=(n_blocks,),
        in_specs=[
            const((h, 128)),
            pl.BlockSpec((_N_BLOCK, in0), lambda i: (i, 0)),
            const((2 * h, in0)),
            const((h, h)),
            const((h, h)),
            const((h, 2 * h)),
            const((h, 1)),
            const((1, 1)),
            const((2 * h, h)),
            const((2 * h, 1)),
            const((1, 1)),
        ],
        out_specs=[
            pl.BlockSpec((2 * h, _N_BLOCK), lambda i: (0, i)),
            pl.BlockSpec((2 * h, _N_BLOCK), lambda i: (0, i)),
            pl.BlockSpec((2 * h, _N_BLOCK), lambda i: (0, i)),
        ],
        out_shape=[
            jax.ShapeDtypeStruct((2 * h, n), jnp.float32),
            jax.ShapeDtypeStruct((2 * h, n), jnp.float32),
            jax.ShapeDtypeStruct((2 * h, n), jnp.float32),
        ],
    )(sums, x_0, w0cat, W1_on, W1_tg, p1_w, p1b, a, p2_w, p2b, inv_e)

    return (on_t.T, pred_t.T, tg_t.T)


# speedup vs baseline: 9.8429x; 1.0202x over previous
import jax
import jax.numpy as jnp
from jax import lax
from jax.experimental import pallas as pl
from jax.experimental.pallas import tpu as pltpu

_E_BLOCK = 80000
_N_BLOCK = 4096


def _colsum_body(x1t_ref, out_ref):
    i = pl.program_id(0)

    @pl.when(i == 0)
    def _init():
        out_ref[...] = jnp.zeros_like(out_ref)

    x = x1t_ref[...]
    xr = x.reshape(x.shape[0], _E_BLOCK // 128, 128)
    out_ref[...] += jnp.sum(xr, axis=1)


def _dense_body(s_ref, x0_ref, w0cat_ref, w1on_ref, w1tg_ref, p1w_ref,
                p1b_ref, a_ref, p2w_ref, p2b_ref, inv_e_ref,
                on_ref, pred_ref, tg_ref):
    def mm(a, b):
        return lax.dot_general(a, b, (((1,), (0,)), ((), ())),
                               preferred_element_type=jnp.float32)

    def mm_t(a, b):
        return lax.dot_general(a, b, (((1,), (1,)), ((), ())),
                               preferred_element_type=jnp.float32)

    m = jnp.sum(s_ref[...], axis=1, keepdims=True) * inv_e_ref[0, 0]
    m_on = mm(w1on_ref[...], m)
    m_tg = mm(w1tg_ref[...], m)

    x0 = x0_ref[...]
    x0_cat_t = mm_t(w0cat_ref[...], x0)
    bn = x0.shape[0]
    h_on_t = jnp.concatenate(
        [x0_cat_t[0:32, :], jnp.broadcast_to(m_on, (32, bn))], axis=0)
    h_tg_t = jnp.concatenate(
        [x0_cat_t[32:64, :], jnp.broadcast_to(m_tg, (32, bn))], axis=0)
    on_ref[...] = h_on_t
    tg_ref[...] = h_tg_t

    z = mm(p1w_ref[...], h_on_t) + p1b_ref[...]
    a = a_ref[0, 0]
    h = jnp.where(z >= 0, z, a * z)
    pred_ref[...] = mm(p2w_ref[...], h) + p2b_ref[...]


def kernel(x_0, x_1, adjacency_0, down_laplacian, up_laplacian,
           W0_on, W1_on, W0_tg, W1_tg, p1_w, p1_b, prelu_a, p2_w, p2_b):
    n, in0 = x_0.shape
    e, h = x_1.shape

    x1t = x_1.T

    sums = pl.pallas_call(
        _colsum_body,
        grid=(e // _E_BLOCK,),
        in_specs=[pl.BlockSpec((h, _E_BLOCK), lambda i: (0, i))],
        out_specs=pl.BlockSpec((h, 128), lambda i: (0, 0)),
        out_shape=jax.ShapeDtypeStruct((h, 128), jnp.float32),
    )(x1t)

    w0cat = jnp.concatenate([W0_on, W0_tg], axis=0)
    p1b = p1_b.reshape(h, 1)
    p2b = p2_b.reshape(2 * h, 1)
    a = jnp.reshape(prelu_a, (1, 1))
    inv_e = jnp.full((1, 1), 1.0 / e, dtype=jnp.float32)

    const = lambda shape: pl.BlockSpec(shape, lambda i: tuple(0 for _ in shape))
    n_blocks = (n + _N_BLOCK - 1) // _N_BLOCK
    on_t, pred_t, tg_t = pl.pallas_call(
        _dense_body,
        grid=(n_blocks,),
        in_specs=[
            const((h, 128)),
            pl.BlockSpec((_N_BLOCK, in0), lambda i: (i, 0)),
            const((2 * h, in0)),
            const((h, h)),
            const((h, h)),
            const((h, 2 * h)),
            const((h, 1)),
            const((1, 1)),
            const((2 * h, h)),
            const((2 * h, 1)),
            const((1, 1)),
        ],
        out_specs=[
            pl.BlockSpec((2 * h, _N_BLOCK), lambda i: (0, i)),
            pl.BlockSpec((2 * h, _N_BLOCK), lambda i: (0, i)),
            pl.BlockSpec((2 * h, _N_BLOCK), lambda i: (0, i)),
        ],
        out_shape=[
            jax.ShapeDtypeStruct((2 * h, n), jnp.float32),
            jax.ShapeDtypeStruct((2 * h, n), jnp.float32),
            jax.ShapeDtypeStruct((2 * h, n), jnp.float32),
        ],
    )(sums, x_0, w0cat, W1_on, W1_tg, p1_w, p1b, a, p2_w, p2b, inv_e)

    return (on_t.T, pred_t.T, tg_t.T)
